# Initial kernel scaffold; baseline (speedup 1.0000x reference)
import jax
import jax.numpy as jnp
from jax.experimental import pallas as pl


def kernel(x, Wq, Wk, Wv, Wproj, gamma, beta):
    raise NotImplementedError("write your pallas kernel here")


# speedup vs baseline: 2.8298x; 2.8298x over previous
import math

import jax
import jax.numpy as jnp
from jax.experimental import pallas as pl

H = 16
D = 64
NH = 2
RADIUS = 6
KBAND = 2 * RADIUS + 1
KPAR = 3
KCHILD = 2
KCAND = KBAND + KPAR + KCHILD


def _prep_body(x_ref, g_ref, b_ref, p1_ref, p2_ref, x0_ref):
    xr = x_ref[0]
    p1b = jnp.max(xr, axis=2)
    p2b = jnp.max(p1b, axis=1)
    m = jnp.mean(p1b, axis=-1, keepdims=True)
    v = jnp.mean((p1b - m) ** 2, axis=-1, keepdims=True)
    x0b = (p1b - m) * jax.lax.rsqrt(v + 1e-5) * g_ref[0] + b_ref[0]
    p1_ref[0] = p1b
    p2_ref[0] = p2b
    x0_ref[0] = x0b


def _matmul_body(a_ref, w_ref, o_ref):
    o_ref[...] = jnp.dot(a_ref[...], w_ref[...],
                         preferred_element_type=jnp.float32)


def _matmul(a, w, bm=512):
    m, k = a.shape
    _, n = w.shape
    return pl.pallas_call(
        _matmul_body,
        grid=(m // bm,),
        in_specs=[
            pl.BlockSpec((bm, k), lambda i: (i, 0)),
            pl.BlockSpec((k, n), lambda i: (0, 0)),
        ],
        out_specs=pl.BlockSpec((bm, n), lambda i: (i, 0)),
        out_shape=jax.ShapeDtypeStruct((m, n), jnp.float32),
    )(a, w)


def _attn_body(q_ref, k0_ref, v0_ref, kp_ref, vp_ref, kc_ref, vc_ref, o_ref):
    q = q_ref[0]
    l, w = q.shape
    lh = l // 2

    row = jax.lax.broadcasted_iota(jnp.int32, (w, NH), 0)
    col = jax.lax.broadcasted_iota(jnp.int32, (w, NH), 1)
    sel = (row // D == col).astype(jnp.float32)

    def head_sums(prod):
        return jnp.dot(prod, sel, preferred_element_type=jnp.float32)

    cand = []
    for o in range(KBAND):
        cand.append(head_sums(q * k0_ref[0, o:o + l]))
    qp = q.reshape(lh, 2, w)
    for s in (1, 0, 2):
        kd = kp_ref[0, s:s + lh]
        prod = (qp * kd[:, None, :]).reshape(l, w)
        cand.append(head_sums(prod))
    kc2 = kc_ref[0].reshape(l, 2, w)
    for c in (0, 1):
        cand.append(head_sums(q * kc2[:, c]))

    inv = 1.0 / math.sqrt(D)
    for h in range(NH):
        scores = jnp.concatenate(
            [sc[:, h:h + 1] for sc in cand], axis=-1) * inv

        iota = jax.lax.broadcasted_iota(jnp.int32, (l, KCAND), 1)
        m1 = jnp.min(scores, axis=-1, keepdims=True)
        i1 = jnp.max(jnp.where(scores == m1, iota, -1), axis=-1,
                     keepdims=True)
        drop1 = iota == i1
        s2 = jnp.where(drop1, jnp.inf, scores)
        m2 = jnp.min(s2, axis=-1, keepdims=True)
        i2 = jnp.max(jnp.where(s2 == m2, iota, -1), axis=-1, keepdims=True)
        keep = jnp.logical_not(drop1 | (iota == i2))

        mx = jnp.max(scores, axis=-1, keepdims=True)
        wgt = jnp.where(keep, jnp.exp(scores - mx), 0.0)
        wgt = wgt / jnp.sum(wgt, axis=-1, keepdims=True)

        c0, c1 = h * D, (h + 1) * D
        out = wgt[:, 0:1] * v0_ref[0, 0:l, c0:c1]
        for o in range(1, KBAND):
            out = out + wgt[:, o:o + 1] * v0_ref[0, o:o + l, c0:c1]
        for j, s in enumerate((1, 0, 2)):
            vps = vp_ref[0, s:s + lh, c0:c1]
            vexp = jnp.broadcast_to(vps[:, None, :], (lh, 2, D)).reshape(l, D)
            out = out + wgt[:, KBAND + j:KBAND + j + 1] * vexp
        vc2 = vc_ref[0].reshape(l, 2, w)
        for c in (0, 1):
            out = out + (wgt[:, KBAND + KPAR + c:KBAND + KPAR + c + 1]
                         * vc2[:, c, c0:c1])
        o_ref[0, :, c0:c1] = out


def kernel(x, Wq, Wk, Wv, Wproj, gamma, beta):
    b, l0, c = x.shape
    l = l0 // 2

    rp = 128
    p1, p2, x0 = pl.pallas_call(
        _prep_body,
        grid=(b, (l0 // 4) // rp),
        in_specs=[
            pl.BlockSpec((1, rp, 2, 2, c), lambda bi, i: (bi, i, 0, 0, 0)),
            pl.BlockSpec((1, c), lambda bi, i: (0, 0)),
            pl.BlockSpec((1, c), lambda bi, i: (0, 0)),
        ],
        out_specs=[
            pl.BlockSpec((1, rp, 2, c), lambda bi, i: (bi, i, 0, 0)),
            pl.BlockSpec((1, rp, c), lambda bi, i: (bi, i, 0)),
            pl.BlockSpec((1, rp, 2, c), lambda bi, i: (bi, i, 0, 0)),
        ],
        out_shape=[
            jax.ShapeDtypeStruct((b, l // 2, 2, c), jnp.float32),
            jax.ShapeDtypeStruct((b, l // 2, c), jnp.float32),
            jax.ShapeDtypeStruct((b, l // 2, 2, c), jnp.float32),
        ],
    )(x.reshape(b, l0 // 4, 2, 2, c), gamma.reshape(1, c),
      beta.reshape(1, c))

    wkv = jnp.concatenate([Wk, Wv], axis=1)
    q2d = _matmul(x0.reshape(b * l, c), Wq)
    kv0 = _matmul(p1.reshape(b * l, c), wkv).reshape(b, l, 2 * c)
    kvp = _matmul(p2.reshape(b * l // 2, c), wkv).reshape(b, l // 2, 2 * c)
    kvc = _matmul(x.reshape(b * l0, c), wkv).reshape(b, l0, 2 * c)

    kv0p = jnp.pad(kv0, ((0, 0), (RADIUS, RADIUS), (0, 0)), mode="edge")
    kvpp = jnp.pad(kvp, ((0, 0), (1, 1), (0, 0)), mode="edge")

    ng = H // NH
    wb = NH * D
    attn = pl.pallas_call(
        _attn_body,
        grid=(b, ng),
        in_specs=[
            pl.BlockSpec((1, l, wb), lambda bi, g: (bi, 0, g)),
            pl.BlockSpec((1, l + 2 * RADIUS, wb), lambda bi, g: (bi, 0, g)),
            pl.BlockSpec((1, l + 2 * RADIUS, wb),
                         lambda bi, g: (bi, 0, ng + g)),
            pl.BlockSpec((1, l // 2 + 2, wb), lambda bi, g: (bi, 0, g)),
            pl.BlockSpec((1, l // 2 + 2, wb), lambda bi, g: (bi, 0, ng + g)),
            pl.BlockSpec((1, 2 * l, wb), lambda bi, g: (bi, 0, g)),
            pl.BlockSpec((1, 2 * l, wb), lambda bi, g: (bi, 0, ng + g)),
        ],
        out_specs=pl.BlockSpec((1, l, wb), lambda bi, g: (bi, 0, g)),
        out_shape=jax.ShapeDtypeStruct((b, l, c), jnp.float32),
    )(q2d.reshape(b, l, c), kv0p, kv0p, kvpp, kvpp, kvc, kvc)

    out = _matmul(attn.reshape(b * l, c), Wproj)
    return out.reshape(b, l, c)
